# Initial kernel scaffold; baseline (speedup 1.0000x reference)
import functools

import jax
import jax.numpy as jnp
from jax import lax
from jax.experimental import pallas as pl
from jax.experimental.pallas import tpu as pltpu
from jax.experimental.pallas import tpu_sc as plsc

NC = 2
NS = 16
LANES = 16


def _sc_body(n_ch, ch, d, b_per_w, b_per_sc,
             ids_hbm, oidx_hbm, w_hbm, out_hbm,
             ids_v, oidx_v, rows_v, zbuf, acc_sh, sem):
    c = lax.axis_index("c")
    s = lax.axis_index("s")
    wid = c * NS + s

    zr = zbuf.shape[0]

    def _zero_row(j, carry):
        for ci in range(d // LANES):
            zbuf[j, pl.ds(ci * LANES, LANES)] = jnp.zeros((LANES,), jnp.float32)
        return carry

    lax.fori_loop(0, zr, _zero_row, 0)
    for r in range(b_per_w // zr):
        pltpu.sync_copy(zbuf, acc_sh.at[pl.ds(s * b_per_w + r * zr, zr)])
    plsc.subcore_barrier()

    pltpu.sync_copy(ids_hbm.at[wid], ids_v)
    pltpu.sync_copy(oidx_hbm.at[wid], oidx_v)

    pltpu.async_copy(w_hbm.at[ids_v.at[0]], rows_v.at[0], sem)

    def _step(j, carry):
        buf = lax.rem(j, 2)
        nxt = lax.rem(j + 1, 2)

        @pl.when(j + 1 < n_ch)
        def _():
            pltpu.async_copy(w_hbm.at[ids_v.at[j + 1]], rows_v.at[nxt], sem)

        pltpu.make_async_copy(w_hbm.at[ids_v.at[j]], rows_v.at[buf], sem).wait()
        pltpu.sync_copy(rows_v.at[buf], acc_sh.at[oidx_v.at[j]], add=True)
        return carry

    lax.fori_loop(0, n_ch, _step, 0)

    plsc.subcore_barrier()
    pltpu.sync_copy(acc_sh.at[pl.ds(s * b_per_w, b_per_w)],
                    out_hbm.at[pl.ds(c * b_per_sc + s * b_per_w, b_per_w)])


def kernel(ids, W):
    b, l = ids.shape
    v, d = W.shape
    nw = NC * NS
    per_w = (b * l) // nw
    ch = 128
    n_ch = per_w // ch
    b_per_w = b // nw
    b_per_sc = b // NC
    zr = 128

    ids_r = ids.reshape(nw, n_ch, ch)
    oidx = (jnp.arange(b * l, dtype=jnp.int32) // l) % b_per_sc
    oidx = oidx.reshape(nw, n_ch, ch)

    mesh = plsc.VectorSubcoreMesh(core_axis_name="c", subcore_axis_name="s",
                                  num_cores=NC, num_subcores=NS)
    run = pl.kernel(
        functools.partial(_sc_body, n_ch, ch, d, b_per_w, b_per_sc),
        out_type=jax.ShapeDtypeStruct((b, d), jnp.float32),
        mesh=mesh,
        scratch_types=[
            pltpu.VMEM((n_ch, ch), jnp.int32),
            pltpu.VMEM((n_ch, ch), jnp.int32),
            pltpu.VMEM((2, ch, d), jnp.float32),
            pltpu.VMEM((zr, d), jnp.float32),
            pltpu.VMEM_SHARED((b_per_sc, d), jnp.float32),
            pltpu.SemaphoreType.DMA,
        ],
    )
    return run(ids_r, oidx, W)


# speedup vs baseline: 2.5863x; 2.5863x over previous
import functools

import jax
import jax.numpy as jnp
from jax import lax
from jax.experimental import pallas as pl
from jax.experimental.pallas import tpu as pltpu
from jax.experimental.pallas import tpu_sc as plsc

NC = 2
NS = 16
LANES = 16


def _sc_body(n_ch, ch, d, b_per_w, b_per_sc,
             ids_hbm, oidx_hbm, w_hbm, out_hbm,
             ids_v, oidx_v, rows_v, zbuf, acc_sh, sem):
    c = lax.axis_index("c")
    s = lax.axis_index("s")
    wid = c * NS + s

    zr = zbuf.shape[0]

    def _zero_row(j, carry):
        for ci in range(d // LANES):
            zbuf[j, pl.ds(ci * LANES, LANES)] = jnp.zeros((LANES,), jnp.float32)
        return carry

    lax.fori_loop(0, zr, _zero_row, 0)
    for r in range(b_per_w // zr):
        pltpu.sync_copy(zbuf, acc_sh.at[pl.ds(s * b_per_w + r * zr, zr)])
    plsc.subcore_barrier()

    pltpu.sync_copy(ids_hbm.at[wid], ids_v)
    pltpu.sync_copy(oidx_hbm.at[wid], oidx_v)

    pltpu.async_copy(w_hbm.at[ids_v.at[0]], rows_v.at[0], sem)

    def _step(j, carry):
        buf = lax.rem(j, 2)
        nxt = lax.rem(j + 1, 2)

        @pl.when(j + 1 < n_ch)
        def _():
            pltpu.async_copy(w_hbm.at[ids_v.at[j + 1]], rows_v.at[nxt], sem)

        pltpu.make_async_copy(w_hbm.at[ids_v.at[j]], rows_v.at[buf], sem).wait()
        pltpu.sync_copy(rows_v.at[buf], acc_sh.at[oidx_v.at[j]], add=True)
        return carry

    lax.fori_loop(0, n_ch, _step, 0)

    plsc.subcore_barrier()
    pltpu.sync_copy(acc_sh.at[pl.ds(s * b_per_w, b_per_w)],
                    out_hbm.at[pl.ds(c * b_per_sc + s * b_per_w, b_per_w)])


def kernel(ids, W):
    b, l = ids.shape
    v, d = W.shape
    nw = NC * NS
    per_w = (b * l) // nw
    ch = 128
    n_ch = per_w // ch
    b_per_w = b // nw
    b_per_sc = b // NC
    zr = 128

    ids_r = ids.reshape(nw, n_ch, ch)
    oidx = (jnp.arange(b * l, dtype=jnp.int32) // l) % b_per_sc
    oidx = oidx.reshape(nw, n_ch, ch)

    mesh = plsc.VectorSubcoreMesh(core_axis_name="c", subcore_axis_name="s",
                                  num_cores=NC, num_subcores=NS)
    run = pl.kernel(
        functools.partial(_sc_body, n_ch, ch, d, b_per_w, b_per_sc),
        out_type=jax.ShapeDtypeStruct((b, d), jnp.float32),
        mesh=mesh,
        compiler_params=pltpu.CompilerParams(use_tc_tiling_on_sc=False),
        scratch_types=[
            pltpu.VMEM((n_ch, ch), jnp.int32),
            pltpu.VMEM((n_ch, ch), jnp.int32),
            pltpu.VMEM((2, ch, d), jnp.float32),
            pltpu.VMEM((zr, d), jnp.float32),
            pltpu.VMEM_SHARED((b_per_sc, d), jnp.float32),
            pltpu.SemaphoreType.DMA,
        ],
    )
    return run(ids_r, oidx, W)


# speedup vs baseline: 2.6432x; 1.0220x over previous
import functools

import jax
import jax.numpy as jnp
from jax import lax
from jax.experimental import pallas as pl
from jax.experimental.pallas import tpu as pltpu
from jax.experimental.pallas import tpu_sc as plsc

NC = 2
NS = 16
LANES = 16


NBUF = 4


def _sc_body(n_ch, ch, d, b_per_w, b_per_sc,
             ids_hbm, oidx_hbm, w_hbm, out_hbm,
             ids_v, oidx_v, rows_v, zbuf, acc_sh, sem_g, sem_s):
    c = lax.axis_index("c")
    s = lax.axis_index("s")
    wid = c * NS + s

    zr = zbuf.shape[0]

    def _zero_row(j, carry):
        for ci in range(d // LANES):
            zbuf[j, pl.ds(ci * LANES, LANES)] = jnp.zeros((LANES,), jnp.float32)
        return carry

    lax.fori_loop(0, zr, _zero_row, 0)
    for r in range(b_per_w // zr):
        pltpu.sync_copy(zbuf, acc_sh.at[pl.ds(s * b_per_w + r * zr, zr)])
    plsc.subcore_barrier()

    pltpu.sync_copy(ids_hbm.at[wid], ids_v)
    pltpu.sync_copy(oidx_hbm.at[wid], oidx_v)

    for p in range(NBUF - 1):
        pltpu.async_copy(w_hbm.at[ids_v.at[p]], rows_v.at[p], sem_g)

    def _step(j, carry):
        buf = lax.rem(j, NBUF)
        pltpu.make_async_copy(w_hbm.at[ids_v.at[j]], rows_v.at[buf], sem_g).wait()
        pltpu.async_copy(rows_v.at[buf], acc_sh.at[oidx_v.at[j]], sem_s, add=True)

        @pl.when(j >= 1)
        def _():
            bp = lax.rem(j - 1, NBUF)
            pltpu.make_async_copy(rows_v.at[bp], acc_sh.at[oidx_v.at[j]],
                                  sem_s).wait()

        @pl.when(j + NBUF - 1 < n_ch)
        def _():
            nxt = lax.rem(j + NBUF - 1, NBUF)
            pltpu.async_copy(w_hbm.at[ids_v.at[j + NBUF - 1]], rows_v.at[nxt],
                             sem_g)

        return carry

    lax.fori_loop(0, n_ch, _step, 0)
    pltpu.make_async_copy(rows_v.at[lax.rem(n_ch - 1, NBUF)],
                          acc_sh.at[oidx_v.at[n_ch - 1]], sem_s).wait()

    plsc.subcore_barrier()
    pltpu.sync_copy(acc_sh.at[pl.ds(s * b_per_w, b_per_w)],
                    out_hbm.at[pl.ds(c * b_per_sc + s * b_per_w, b_per_w)])


def kernel(ids, W):
    b, l = ids.shape
    v, d = W.shape
    nw = NC * NS
    per_w = (b * l) // nw
    ch = 128
    n_ch = per_w // ch
    b_per_w = b // nw
    b_per_sc = b // NC
    zr = 128

    ids_r = ids.reshape(nw, n_ch, ch)
    oidx = (jnp.arange(b * l, dtype=jnp.int32) // l) % b_per_sc
    oidx = oidx.reshape(nw, n_ch, ch)

    mesh = plsc.VectorSubcoreMesh(core_axis_name="c", subcore_axis_name="s",
                                  num_cores=NC, num_subcores=NS)
    run = pl.kernel(
        functools.partial(_sc_body, n_ch, ch, d, b_per_w, b_per_sc),
        out_type=jax.ShapeDtypeStruct((b, d), jnp.float32),
        mesh=mesh,
        compiler_params=pltpu.CompilerParams(use_tc_tiling_on_sc=False),
        scratch_types=[
            pltpu.VMEM((n_ch, ch), jnp.int32),
            pltpu.VMEM((n_ch, ch), jnp.int32),
            pltpu.VMEM((NBUF, ch, d), jnp.float32),
            pltpu.VMEM((zr, d), jnp.float32),
            pltpu.VMEM_SHARED((b_per_sc, d), jnp.float32),
            pltpu.SemaphoreType.DMA,
            pltpu.SemaphoreType.DMA,
        ],
    )
    return run(ids_r, oidx, W)


# speedup vs baseline: 2.7842x; 1.0533x over previous
import functools

import jax
import jax.numpy as jnp
from jax import lax
from jax.experimental import pallas as pl
from jax.experimental.pallas import tpu as pltpu
from jax.experimental.pallas import tpu_sc as plsc

NC = 2
NS = 16
LANES = 16
RPC = 2
NBUF = 4


def _sc_body(n_ch, l, d, b_per_w,
             ids_hbm, w_hbm, out_hbm, ids_v, rows_v, obuf, sem_g, sem_o):
    c = lax.axis_index("c")
    s = lax.axis_index("s")
    wid = c * NS + s
    ch = RPC * l
    nsub = d // LANES

    pltpu.sync_copy(ids_hbm.at[wid], ids_v)

    for p in range(NBUF - 1):
        pltpu.async_copy(w_hbm.at[ids_v.at[p]], rows_v.at[p], sem_g)

    def _step(j, carry):
        buf = lax.rem(j, NBUF)
        pltpu.make_async_copy(w_hbm.at[ids_v.at[j]], rows_v.at[buf],
                              sem_g).wait()

        @pl.when(j + NBUF - 1 < n_ch)
        def _():
            nxt = lax.rem(j + NBUF - 1, NBUF)
            pltpu.async_copy(w_hbm.at[ids_v.at[j + NBUF - 1]], rows_v.at[nxt],
                             sem_g)

        for r in range(RPC):
            accs = [rows_v[buf, r * l, pl.ds(ci * LANES, LANES)]
                    for ci in range(nsub)]
            for k in range(1, l):
                for ci in range(nsub):
                    accs[ci] = accs[ci] + rows_v[buf, r * l + k,
                                                 pl.ds(ci * LANES, LANES)]
            for ci in range(nsub):
                obuf[RPC * j + r, pl.ds(ci * LANES, LANES)] = accs[ci]
        return carry

    lax.fori_loop(0, n_ch, _step, 0)
    pltpu.sync_copy(obuf, out_hbm.at[pl.ds(wid * b_per_w, b_per_w)])


def kernel(ids, W):
    b, l = ids.shape
    v, d = W.shape
    nw = NC * NS
    per_w = (b * l) // nw
    ch = RPC * l
    n_ch = per_w // ch
    b_per_w = b // nw

    ids_r = ids.reshape(nw, n_ch, ch)

    mesh = plsc.VectorSubcoreMesh(core_axis_name="c", subcore_axis_name="s",
                                  num_cores=NC, num_subcores=NS)
    run = pl.kernel(
        functools.partial(_sc_body, n_ch, l, d, b_per_w),
        out_type=jax.ShapeDtypeStruct((b, d), jnp.float32),
        mesh=mesh,
        compiler_params=pltpu.CompilerParams(use_tc_tiling_on_sc=False),
        scratch_types=[
            pltpu.VMEM((n_ch, ch), jnp.int32),
            pltpu.VMEM((NBUF, ch, d), jnp.float32),
            pltpu.VMEM((b_per_w, d), jnp.float32),
            pltpu.SemaphoreType.DMA,
            pltpu.SemaphoreType.DMA,
        ],
    )
    return run(ids_r, W)
